# initial kernel scaffold (unmeasured)
import jax
import jax.numpy as jnp
from jax import lax
from jax.experimental import pallas as pl
from jax.experimental.pallas import tpu as pltpu


def _exchange_body(send_ref, out_ref, send_sem, recv_sem):
    my_x = lax.axis_index("x")
    my_y = lax.axis_index("y")
    my_z = lax.axis_index("z")
    rdma = pltpu.make_async_remote_copy(
        src_ref=send_ref,
        dst_ref=out_ref,
        send_sem=send_sem,
        recv_sem=recv_sem,
        device_id=(my_x, 1 - my_y, my_z),
        device_id_type=pl.DeviceIdType.MESH,
    )
    rdma.start()
    rdma.wait()


def kernel(x, dy):
    m, d = x.shape
    _, n = dy.shape
    half = d // 2

    my_y = lax.axis_index("y")
    partial = jnp.dot(x.T, dy, preferred_element_type=jnp.float32)
    keep = lax.dynamic_slice_in_dim(partial, my_y * half, half, axis=0)
    send = lax.dynamic_slice_in_dim(partial, (1 - my_y) * half, half, axis=0)

    recv = pl.pallas_call(
        _exchange_body,
        out_shape=jax.ShapeDtypeStruct((half, n), jnp.float32),
        in_specs=[pl.BlockSpec(memory_space=pltpu.ANY)],
        out_specs=pl.BlockSpec(memory_space=pltpu.ANY),
        scratch_shapes=[pltpu.SemaphoreType.DMA, pltpu.SemaphoreType.DMA],
        compiler_params=pltpu.CompilerParams(collective_id=0),
    )(send)
    return keep + recv


# baseline (device time: 1245557 ns/iter reference)
import jax
import jax.numpy as jnp
from jax import lax
from jax.experimental import pallas as pl
from jax.experimental.pallas import tpu as pltpu


def _exchange_body(send_ref, out_ref, send_sem, recv_sem):
    my_x = lax.axis_index("x")
    my_y = lax.axis_index("y")
    my_z = lax.axis_index("z")
    barrier_sem = pltpu.get_barrier_semaphore()
    pl.semaphore_signal(
        barrier_sem,
        inc=1,
        device_id=(my_x, 1 - my_y, my_z),
        device_id_type=pl.DeviceIdType.MESH,
    )
    pl.semaphore_wait(barrier_sem, 1)
    rdma = pltpu.make_async_remote_copy(
        src_ref=send_ref,
        dst_ref=out_ref,
        send_sem=send_sem,
        recv_sem=recv_sem,
        device_id=(my_x, 1 - my_y, my_z),
        device_id_type=pl.DeviceIdType.MESH,
    )
    rdma.start()
    rdma.wait()


def kernel(x, dy):
    m, d = x.shape
    _, n = dy.shape
    half = d // 2

    my_y = lax.axis_index("y")
    partial = jnp.dot(x.T, dy, preferred_element_type=jnp.float32)
    keep = lax.dynamic_slice_in_dim(partial, my_y * half, half, axis=0)
    send = lax.dynamic_slice_in_dim(partial, (1 - my_y) * half, half, axis=0)

    recv = pl.pallas_call(
        _exchange_body,
        out_shape=jax.ShapeDtypeStruct((half, n), jnp.float32),
        in_specs=[pl.BlockSpec(memory_space=pl.ANY)],
        out_specs=pl.BlockSpec(memory_space=pl.ANY),
        scratch_shapes=[pltpu.SemaphoreType.DMA, pltpu.SemaphoreType.DMA],
        compiler_params=pltpu.CompilerParams(collective_id=0),
    )(send)
    return keep + recv


# device time: 851257 ns/iter; 1.4632x vs baseline; 1.4632x over previous
import jax
import jax.numpy as jnp
from jax import lax
from jax.experimental import pallas as pl
from jax.experimental.pallas import tpu as pltpu

CH = 8
W = 4096 // CH
HALF_ROWS = 2048


def _mm_body(x_ref, dy_ref, o_ref):
    k = pl.program_id(1)

    @pl.when(k == 0)
    def _():
        o_ref[...] = jnp.zeros_like(o_ref)

    o_ref[...] += lax.dot_general(
        x_ref[...],
        dy_ref[...],
        dimension_numbers=(((0,), (0,)), ((), ())),
        preferred_element_type=jnp.float32,
    )


def _gemm(x, dy, nc=1024, mk=256):
    m, d = x.shape
    _, n = dy.shape
    return pl.pallas_call(
        _mm_body,
        grid=(n // nc, m // mk),
        in_specs=[
            pl.BlockSpec((mk, d), lambda j, k: (k, 0)),
            pl.BlockSpec((mk, nc), lambda j, k: (k, j)),
        ],
        out_specs=pl.BlockSpec((d, nc), lambda j, k: (0, j)),
        out_shape=jax.ShapeDtypeStruct((d, n), jnp.float32),
        compiler_params=pltpu.CompilerParams(
            dimension_semantics=("parallel", "arbitrary"),
            vmem_limit_bytes=60 * 1024 * 1024,
        ),
    )(x, dy)


def _rs_body(
    partial_ref,
    out_ref,
    recv_ref,
    y_send_sems,
    y_recv_sems,
    x_send_sems,
    x_recv_sems,
    stage_a,
    stage_b,
    cp_sem_a,
    cp_sem_b,
    cp_sem_o,
):
    my_x = lax.axis_index("x")
    my_y = lax.axis_index("y")
    my_z = lax.axis_index("z")
    y_partner = (my_x, 1 - my_y, my_z)
    x_nbr = (1 - my_x, my_y, my_z)

    barrier = pltpu.get_barrier_semaphore()
    for nbr in (y_partner, x_nbr):
        pl.semaphore_signal(
            barrier, inc=1, device_id=nbr, device_id_type=pl.DeviceIdType.MESH
        )
    pl.semaphore_wait(barrier, 2)

    row_keep = my_y * HALF_ROWS
    row_send = (1 - my_y) * HALF_ROWS
    col_mine = my_x * (CH * W)
    col_other = (1 - my_x) * (CH * W)

    def y_send(c):
        col = col_mine + c * W
        return pltpu.make_async_remote_copy(
            src_ref=partial_ref.at[pl.ds(row_send, HALF_ROWS), pl.ds(col, W)],
            dst_ref=recv_ref.at[:, pl.ds(col, W)],
            send_sem=y_send_sems.at[c],
            recv_sem=y_recv_sems.at[c],
            device_id=y_partner,
            device_id_type=pl.DeviceIdType.MESH,
        )

    def x_fwd(c):
        col = col_mine + c * W
        return pltpu.make_async_remote_copy(
            src_ref=recv_ref.at[:, pl.ds(col, W)],
            dst_ref=recv_ref.at[:, pl.ds(col, W)],
            send_sem=x_send_sems.at[c],
            recv_sem=x_recv_sems.at[c],
            device_id=x_nbr,
            device_id_type=pl.DeviceIdType.MESH,
        )

    def x_recv(c):
        col = col_other + c * W
        return pltpu.make_async_remote_copy(
            src_ref=recv_ref.at[:, pl.ds(col, W)],
            dst_ref=recv_ref.at[:, pl.ds(col, W)],
            send_sem=x_send_sems.at[c],
            recv_sem=x_recv_sems.at[c],
            device_id=x_nbr,
            device_id_type=pl.DeviceIdType.MESH,
        )

    def add_chunk(col):
        cp_a = pltpu.make_async_copy(
            partial_ref.at[pl.ds(row_keep, HALF_ROWS), pl.ds(col, W)],
            stage_a,
            cp_sem_a,
        )
        cp_b = pltpu.make_async_copy(
            recv_ref.at[:, pl.ds(col, W)], stage_b, cp_sem_b
        )
        cp_a.start()
        cp_b.start()
        cp_a.wait()
        cp_b.wait()
        stage_a[...] += stage_b[...]
        cp_o = pltpu.make_async_copy(
            stage_a, out_ref.at[:, pl.ds(col, W)], cp_sem_o
        )
        cp_o.start()
        cp_o.wait()

    y_rdmas = [y_send(c) for c in range(CH)]
    for r in y_rdmas:
        r.start()

    fwds = []
    for c in range(CH):
        y_rdmas[c].wait_recv()
        f = x_fwd(c)
        f.start()
        fwds.append(f)
        add_chunk(col_mine + c * W)

    for c in range(CH):
        x_recv(c).wait_recv()
        add_chunk(col_other + c * W)

    for c in range(CH):
        y_rdmas[c].wait_send()
        fwds[c].wait_send()


def kernel(x, dy):
    m, d = x.shape
    _, n = dy.shape

    partial = _gemm(x, dy)

    out, _ = pl.pallas_call(
        _rs_body,
        out_shape=(
            jax.ShapeDtypeStruct((HALF_ROWS, n), jnp.float32),
            jax.ShapeDtypeStruct((HALF_ROWS, n), jnp.float32),
        ),
        in_specs=[pl.BlockSpec(memory_space=pl.ANY)],
        out_specs=(
            pl.BlockSpec(memory_space=pl.ANY),
            pl.BlockSpec(memory_space=pl.ANY),
        ),
        scratch_shapes=[
            pltpu.SemaphoreType.DMA((CH,)),
            pltpu.SemaphoreType.DMA((CH,)),
            pltpu.SemaphoreType.DMA((CH,)),
            pltpu.SemaphoreType.DMA((CH,)),
            pltpu.MemorySpace.VMEM((HALF_ROWS, W), jnp.float32),
            pltpu.MemorySpace.VMEM((HALF_ROWS, W), jnp.float32),
            pltpu.SemaphoreType.DMA,
            pltpu.SemaphoreType.DMA,
            pltpu.SemaphoreType.DMA,
        ],
        compiler_params=pltpu.CompilerParams(collective_id=0),
    )(partial)
    return out


# device time: 650288 ns/iter; 1.9154x vs baseline; 1.3090x over previous
import jax
import jax.numpy as jnp
from jax import lax
from jax.experimental import pallas as pl
from jax.experimental.pallas import tpu as pltpu

CH = 8
W = 4096 // CH
HALF_ROWS = 2048


def _mm_body(x_ref, dy_ref, o_ref, acc_ref):
    k = pl.program_id(1)

    @pl.when(k == 0)
    def _():
        acc_ref[...] = jnp.zeros_like(acc_ref)

    acc_ref[...] += lax.dot_general(
        x_ref[...],
        dy_ref[...],
        dimension_numbers=(((0,), (0,)), ((), ())),
        preferred_element_type=jnp.float32,
    )

    @pl.when(k == pl.num_programs(1) - 1)
    def _():
        o_ref[...] = acc_ref[...].astype(jnp.bfloat16)


def _gemm(x, dy, nc=1024, mk=256):
    m, d = x.shape
    _, n = dy.shape
    return pl.pallas_call(
        _mm_body,
        grid=(n // nc, m // mk),
        in_specs=[
            pl.BlockSpec((mk, d), lambda j, k: (k, 0)),
            pl.BlockSpec((mk, nc), lambda j, k: (k, j)),
        ],
        out_specs=pl.BlockSpec((d, nc), lambda j, k: (0, j)),
        out_shape=jax.ShapeDtypeStruct((d, n), jnp.bfloat16),
        scratch_shapes=[pltpu.MemorySpace.VMEM((d, nc), jnp.float32)],
        compiler_params=pltpu.CompilerParams(
            dimension_semantics=("parallel", "arbitrary"),
            vmem_limit_bytes=60 * 1024 * 1024,
        ),
    )(x, dy)


def _rs_body(
    partial_ref,
    out_ref,
    recv_ref,
    y_send_sems,
    y_recv_sems,
    x_send_sems,
    x_recv_sems,
    stage_a,
    stage_b,
    stage_o,
    cp_sem_a,
    cp_sem_b,
    cp_sem_o,
):
    my_x = lax.axis_index("x")
    my_y = lax.axis_index("y")
    my_z = lax.axis_index("z")
    y_partner = (my_x, 1 - my_y, my_z)
    x_nbr = (1 - my_x, my_y, my_z)

    barrier = pltpu.get_barrier_semaphore()
    for nbr in (y_partner, x_nbr):
        pl.semaphore_signal(
            barrier, inc=1, device_id=nbr, device_id_type=pl.DeviceIdType.MESH
        )
    pl.semaphore_wait(barrier, 2)

    row_keep = my_y * HALF_ROWS
    row_send = (1 - my_y) * HALF_ROWS
    col_mine = my_x * (CH * W)
    col_other = (1 - my_x) * (CH * W)

    def y_send(c):
        col = col_mine + c * W
        return pltpu.make_async_remote_copy(
            src_ref=partial_ref.at[pl.ds(row_send, HALF_ROWS), pl.ds(col, W)],
            dst_ref=recv_ref.at[:, pl.ds(col, W)],
            send_sem=y_send_sems.at[c],
            recv_sem=y_recv_sems.at[c],
            device_id=y_partner,
            device_id_type=pl.DeviceIdType.MESH,
        )

    def x_fwd(c):
        col = col_mine + c * W
        return pltpu.make_async_remote_copy(
            src_ref=recv_ref.at[:, pl.ds(col, W)],
            dst_ref=recv_ref.at[:, pl.ds(col, W)],
            send_sem=x_send_sems.at[c],
            recv_sem=x_recv_sems.at[c],
            device_id=x_nbr,
            device_id_type=pl.DeviceIdType.MESH,
        )

    def x_recv(c):
        col = col_other + c * W
        return pltpu.make_async_remote_copy(
            src_ref=recv_ref.at[:, pl.ds(col, W)],
            dst_ref=recv_ref.at[:, pl.ds(col, W)],
            send_sem=x_send_sems.at[c],
            recv_sem=x_recv_sems.at[c],
            device_id=x_nbr,
            device_id_type=pl.DeviceIdType.MESH,
        )

    def add_chunk(col):
        cp_a = pltpu.make_async_copy(
            partial_ref.at[pl.ds(row_keep, HALF_ROWS), pl.ds(col, W)],
            stage_a,
            cp_sem_a,
        )
        cp_b = pltpu.make_async_copy(
            recv_ref.at[:, pl.ds(col, W)], stage_b, cp_sem_b
        )
        cp_a.start()
        cp_b.start()
        cp_a.wait()
        cp_b.wait()
        stage_o[...] = stage_a[...].astype(jnp.float32) + stage_b[...].astype(
            jnp.float32
        )
        cp_o = pltpu.make_async_copy(
            stage_o, out_ref.at[:, pl.ds(col, W)], cp_sem_o
        )
        cp_o.start()
        cp_o.wait()

    y_rdmas = [y_send(c) for c in range(CH)]
    for r in y_rdmas:
        r.start()

    fwds = []
    for c in range(CH):
        y_rdmas[c].wait_recv()
        f = x_fwd(c)
        f.start()
        fwds.append(f)
        add_chunk(col_mine + c * W)

    for c in range(CH):
        x_recv(c).wait_recv()
        add_chunk(col_other + c * W)

    for c in range(CH):
        y_rdmas[c].wait_send()
        fwds[c].wait_send()


def kernel(x, dy):
    m, d = x.shape
    _, n = dy.shape

    partial = _gemm(x, dy)

    out, _ = pl.pallas_call(
        _rs_body,
        out_shape=(
            jax.ShapeDtypeStruct((HALF_ROWS, n), jnp.float32),
            jax.ShapeDtypeStruct((HALF_ROWS, n), jnp.bfloat16),
        ),
        in_specs=[pl.BlockSpec(memory_space=pl.ANY)],
        out_specs=(
            pl.BlockSpec(memory_space=pl.ANY),
            pl.BlockSpec(memory_space=pl.ANY),
        ),
        scratch_shapes=[
            pltpu.SemaphoreType.DMA((CH,)),
            pltpu.SemaphoreType.DMA((CH,)),
            pltpu.SemaphoreType.DMA((CH,)),
            pltpu.SemaphoreType.DMA((CH,)),
            pltpu.MemorySpace.VMEM((HALF_ROWS, W), jnp.bfloat16),
            pltpu.MemorySpace.VMEM((HALF_ROWS, W), jnp.bfloat16),
            pltpu.MemorySpace.VMEM((HALF_ROWS, W), jnp.float32),
            pltpu.SemaphoreType.DMA,
            pltpu.SemaphoreType.DMA,
            pltpu.SemaphoreType.DMA,
        ],
        compiler_params=pltpu.CompilerParams(collective_id=0),
    )(partial)
    return out
